# baseline (speedup 1.0000x reference)
import functools
import math

import jax
import jax.numpy as jnp
from jax import lax
from jax.experimental import pallas as pl
from jax.experimental.pallas import tpu as pltpu
from jax.experimental.pallas import tpu_sc as plsc

D_MODEL = 64
SCALE = math.sqrt(D_MODEL)

NUM_CORES = 2
NUM_SUBCORES = 16
NUM_WORKERS = NUM_CORES * NUM_SUBCORES
LANES = 16

CHUNK = 1024


def _emb_kernel(idx_hbm, table_hbm, out_hbm, idx_v, rows_v, sem):
    wid = lax.axis_index("s") * NUM_CORES + lax.axis_index("c")
    b_per_w = idx_hbm.shape[0] // NUM_WORKERS
    n_chunks = b_per_w // CHUNK
    base = wid * b_per_w

    def step(c, _):
        off = base + c * CHUNK
        pltpu.sync_copy(idx_hbm.at[pl.ds(off, CHUNK)], idx_v)
        pltpu.async_copy(table_hbm.at[idx_v], rows_v, sem).wait()

        def scale_row(r, _):
            for j in range(D_MODEL // LANES):
                sl = pl.ds(j * LANES, LANES)
                rows_v[r, sl] = rows_v[r, sl] * SCALE
            return 0

        lax.fori_loop(0, CHUNK, scale_row, 0)
        pltpu.sync_copy(rows_v, out_hbm.at[pl.ds(off, CHUNK)])
        return 0

    lax.fori_loop(0, n_chunks, step, 0)


def kernel(x, table):
    batch, seq = x.shape
    n = batch * seq
    idx = x.reshape(n)

    mesh = plsc.VectorSubcoreMesh(core_axis_name="c", subcore_axis_name="s")
    run = pl.kernel(
        _emb_kernel,
        out_type=jax.ShapeDtypeStruct((n, D_MODEL), jnp.float32),
        mesh=mesh,
        scratch_types=[
            pltpu.VMEM((CHUNK,), jnp.int32),
            pltpu.VMEM((CHUNK, D_MODEL), jnp.float32),
            pltpu.SemaphoreType.DMA,
        ],
        compiler_params=pltpu.CompilerParams(use_tc_tiling_on_sc=False),
    )
    out = run(idx, table)
    return out.reshape(batch, seq, D_MODEL)


# speedup vs baseline: 1.1088x; 1.1088x over previous
import functools
import math

import jax
import jax.numpy as jnp
from jax import lax
from jax.experimental import pallas as pl
from jax.experimental.pallas import tpu as pltpu
from jax.experimental.pallas import tpu_sc as plsc

D_MODEL = 64
SCALE = math.sqrt(D_MODEL)

NUM_CORES = 2
NUM_SUBCORES = 16
NUM_WORKERS = NUM_CORES * NUM_SUBCORES
LANES = 16

CHUNK = 320
NBUF = 4


def _emb_kernel(idx_hbm, table_hbm, out_hbm, idx_v, rows, gsems, osems):
    wid = lax.axis_index("s") * NUM_CORES + lax.axis_index("c")
    b_per_w = idx_hbm.shape[0] // NUM_WORKERS
    n_chunks = b_per_w // CHUNK
    base = wid * b_per_w

    def idx_slice(c):
        return idx_v.at[pl.ds(c * CHUNK, CHUNK)]

    def gather(c, b):
        pltpu.async_copy(table_hbm.at[idx_slice(c)], rows[b], gsems[b])

    def gather_wait(b):
        pltpu.make_async_copy(table_hbm.at[idx_slice(0)], rows[b], gsems[b]).wait()

    def writeback(c, b):
        pltpu.async_copy(
            rows[b], out_hbm.at[pl.ds(base + c * CHUNK, CHUNK)], osems[b]
        )

    def writeback_wait(b):
        pltpu.make_async_copy(
            rows[b], out_hbm.at[pl.ds(base, CHUNK)], osems[b]
        ).wait()

    pltpu.sync_copy(idx_hbm.at[pl.ds(base, b_per_w)], idx_v)
    gather(0, 0)

    def group(g, _):
        for b in range(NBUF):
            c = g * NBUF + b
            b_next = (b + 1) % NBUF

            @pl.when(c >= NBUF - 1)
            def _():
                writeback_wait(b_next)

            @pl.when(c + 1 < n_chunks)
            def _():
                gather(c + 1, b_next)

            gather_wait(b)

            @plsc.parallel_loop(0, CHUNK, step=1, unroll=8)
            def _(r):
                for j in range(D_MODEL // LANES):
                    sl = pl.ds(j * LANES, LANES)
                    rows[b][r, sl] = rows[b][r, sl] * SCALE

            writeback(c, b)
        return 0

    lax.fori_loop(0, n_chunks // NBUF, group, 0)

    for k in range(1, NBUF):
        writeback_wait((n_chunks - k) % NBUF)


def kernel(x, table):
    batch, seq = x.shape
    n = batch * seq
    idx = x.reshape(n)

    mesh = plsc.VectorSubcoreMesh(core_axis_name="c", subcore_axis_name="s")
    run = pl.kernel(
        _emb_kernel,
        out_type=jax.ShapeDtypeStruct((n, D_MODEL), jnp.float32),
        mesh=mesh,
        scratch_types=[
            pltpu.VMEM((n // NUM_WORKERS,), jnp.int32),
            [pltpu.VMEM((CHUNK, D_MODEL), jnp.float32) for _ in range(NBUF)],
            [pltpu.SemaphoreType.DMA for _ in range(NBUF)],
            [pltpu.SemaphoreType.DMA for _ in range(NBUF)],
        ],
        compiler_params=pltpu.CompilerParams(use_tc_tiling_on_sc=False),
    )
    out = run(idx, table)
    return out.reshape(batch, seq, D_MODEL)
